# baseline (speedup 1.0000x reference)
import functools

import jax
import jax.numpy as jnp
from jax import lax
from jax.experimental import pallas as pl
from jax.experimental.pallas import tpu as pltpu
from jax.experimental.pallas import tpu_sc as plsc

_LANES = 16


def _sqrt16(x):
    i = lax.bitcast_convert_type(x, jnp.int32)
    i = 0x5F3759DF - lax.shift_right_logical(i, 1)
    y = lax.bitcast_convert_type(i, jnp.float32)
    xh = x * 0.5
    y = y * (1.5 - xh * y * y)
    y = y * (1.5 - xh * y * y)
    return x * y


def _edge(p, q):
    d0 = p[0] - q[0]
    d1 = p[1] - q[1]
    d2 = p[2] - q[2]
    return _sqrt16(d0 * d0 + d1 * d1 + d2 * d2 + 1e-12)


def kernel(coord_out, coord_gt, face):
    B, V, _ = coord_out.shape
    Fn = face.shape[0]
    RW = 3 * V

    info = plsc.get_sparse_core_info()
    NW = info.num_cores * info.num_subcores
    NC = info.num_cores
    rows_w = B // NW
    C = 16
    n_chunks = rows_w // C
    GROUPS = Fn // _LANES

    mesh = plsc.VectorSubcoreMesh(core_axis_name="c", subcore_axis_name="s")

    @functools.partial(
        pl.kernel,
        mesh=mesh,
        out_type=jax.ShapeDtypeStruct((NW, _LANES), jnp.float32),
        compiler_params=pltpu.CompilerParams(needs_layout_passes=False),
        scratch_types=[
            pltpu.VMEM((Fn * 3,), jnp.int32),
            pltpu.VMEM((C * RW,), jnp.float32),
            pltpu.VMEM((C * RW,), jnp.float32),
            pltpu.VMEM((_LANES,), jnp.float32),
        ],
    )
    def edge_loss(co_hbm, cg_hbm, face_hbm, out_hbm, face_v, bo, bg, accv):
        wid = lax.axis_index("s") * NC + lax.axis_index("c")
        pltpu.sync_copy(face_hbm, face_v)
        iota = lax.iota(jnp.int32, _LANES)

        base = []
        for g in range(GROUPS):
            rows = g * _LANES + iota
            segs = []
            for seg in range(3):
                vid = plsc.load_gather(face_v, [rows * 3 + seg])
                v3 = vid * 3
                segs.append([v3, v3 + 1, v3 + 2])
            base.append(segs)

        def chunk_body(i, acc):
            off = (wid * rows_w + i * C) * RW
            pltpu.sync_copy(co_hbm.at[pl.ds(off, C * RW)], bo)
            pltpu.sync_copy(cg_hbm.at[pl.ds(off, C * RW)], bg)

            def row_body(r, acc):
                ro = r * RW
                for g in range(GROUPS):
                    idx = [[base[g][s][c] + ro for c in range(3)]
                           for s in range(3)]
                    po = [[plsc.load_gather(bo, [idx[s][c]]) for c in range(3)]
                          for s in range(3)]
                    pg = [[plsc.load_gather(bg, [idx[s][c]]) for c in range(3)]
                          for s in range(3)]
                    for a, b in ((0, 1), (0, 2), (1, 2)):
                        d_o = _edge(po[a], po[b])
                        d_g = _edge(pg[a], pg[b])
                        acc = acc + jnp.abs(d_o - d_g)
                return acc

            return lax.fori_loop(0, C, row_body, acc)

        acc = lax.fori_loop(0, n_chunks, chunk_body,
                            jnp.zeros((_LANES,), jnp.float32))
        accv[...] = acc
        pltpu.sync_copy(accv, out_hbm.at[wid])

    partial = edge_loss(coord_out.reshape(-1), coord_gt.reshape(-1),
                        face.reshape(-1))
    return jnp.sum(partial) / (B * 3 * Fn)


# speedup vs baseline: 125.6516x; 125.6516x over previous
import functools

import jax
import jax.numpy as jnp
from jax import lax
from jax.experimental import pallas as pl
from jax.experimental.pallas import tpu as pltpu
from jax.experimental.pallas import tpu_sc as plsc

_LANES = 16


def _sqrt16(x):
    i = lax.bitcast_convert_type(x, jnp.int32)
    i = 0x5F3759DF - lax.shift_right_logical(i, 1)
    y = lax.bitcast_convert_type(i, jnp.float32)
    xh = x * 0.5
    y = y * (1.5 - xh * y * y)
    y = y * (1.5 - xh * y * y)
    return x * y


def _edge(p, q):
    d0 = p[0] - q[0]
    d1 = p[1] - q[1]
    d2 = p[2] - q[2]
    return _sqrt16(d0 * d0 + d1 * d1 + d2 * d2 + 1e-12)


def kernel(coord_out, coord_gt, face):
    B, V, _ = coord_out.shape
    Fn = face.shape[0]

    info = plsc.get_sparse_core_info()
    NW = info.num_cores * info.num_subcores
    NC = info.num_cores
    rows_w = B // NW
    C = 16
    n_chunks = rows_w // C
    GROUPS = Fn // _LANES

    mesh = plsc.VectorSubcoreMesh(core_axis_name="c", subcore_axis_name="s")

    @functools.partial(
        pl.kernel,
        mesh=mesh,
        out_type=jax.ShapeDtypeStruct((NW, _LANES), jnp.float32),
        compiler_params=pltpu.CompilerParams(needs_layout_passes=False),
        scratch_types=[
            pltpu.VMEM((Fn * 3,), jnp.int32),
            [[pltpu.VMEM((C, V), jnp.float32) for _ in range(3)]
             for _ in range(2)],
            pltpu.VMEM((_LANES,), jnp.float32),
        ],
    )
    def edge_loss(co_hbm, cg_hbm, face_hbm, out_hbm, face_v, bufs, accv):
        wid = lax.axis_index("s") * NC + lax.axis_index("c")
        pltpu.sync_copy(face_hbm, face_v)
        iota = lax.iota(jnp.int32, _LANES)

        vids = []
        for g in range(GROUPS):
            rows = g * _LANES + iota
            vids.append([plsc.load_gather(face_v, [rows * 3 + seg])
                         for seg in range(3)])

        def chunk_body(i, acc):
            r0 = wid * rows_w + i * C
            for a, hbm in enumerate((co_hbm, cg_hbm)):
                for p in range(3):
                    pltpu.sync_copy(hbm.at[p, pl.ds(r0, C), :], bufs[a][p])

            def row_body(r, acc):
                rsp = jnp.full((_LANES,), r, jnp.int32)
                for g in range(GROUPS):
                    pts = [[[plsc.load_gather(bufs[a][p], [rsp, vids[g][s]])
                             for p in range(3)]
                            for s in range(3)]
                           for a in range(2)]
                    for s0, s1 in ((0, 1), (0, 2), (1, 2)):
                        d_o = _edge(pts[0][s0], pts[0][s1])
                        d_g = _edge(pts[1][s0], pts[1][s1])
                        acc = acc + jnp.abs(d_o - d_g)
                return acc

            return lax.fori_loop(0, C, row_body, acc)

        acc = lax.fori_loop(0, n_chunks, chunk_body,
                            jnp.zeros((_LANES,), jnp.float32))
        accv[...] = acc
        pltpu.sync_copy(accv, out_hbm.at[wid])

    partial = edge_loss(coord_out.transpose(2, 0, 1),
                        coord_gt.transpose(2, 0, 1),
                        face.reshape(-1))
    return jnp.sum(partial) / (B * 3 * Fn)


# speedup vs baseline: 297.8342x; 2.3703x over previous
import functools

import jax
import jax.numpy as jnp
from jax import lax
from jax.experimental import pallas as pl
from jax.experimental.pallas import tpu as pltpu
from jax.experimental.pallas import tpu_sc as plsc

_LANES = 16


def _sqrt16(x):
    i = lax.bitcast_convert_type(x, jnp.int32)
    i = 0x5F3759DF - lax.shift_right_logical(i, 1)
    y = lax.bitcast_convert_type(i, jnp.float32)
    y = y * (1.5 - (x * 0.5) * y * y)
    return x * y


def _edge(p, q):
    d0 = p[0] - q[0]
    d1 = p[1] - q[1]
    d2 = p[2] - q[2]
    return _sqrt16(d0 * d0 + d1 * d1 + d2 * d2 + 1e-12)


def kernel(coord_out, coord_gt, face):
    B, V, _ = coord_out.shape
    Fn = face.shape[0]

    info = plsc.get_sparse_core_info()
    NW = info.num_cores * info.num_subcores
    NC = info.num_cores
    rows_w = B // NW
    C = 16
    n_chunks = rows_w // C
    GROUPS = Fn // _LANES

    mesh = plsc.VectorSubcoreMesh(core_axis_name="c", subcore_axis_name="s")

    @functools.partial(
        pl.kernel,
        mesh=mesh,
        out_type=jax.ShapeDtypeStruct((NW, _LANES), jnp.float32),
        compiler_params=pltpu.CompilerParams(needs_layout_passes=False),
        scratch_types=[
            pltpu.VMEM((Fn * 3,), jnp.int32),
            [[[pltpu.VMEM((C, V), jnp.float32) for _ in range(3)]
              for _ in range(2)]
             for _ in range(2)],
            [pltpu.SemaphoreType.DMA for _ in range(2)],
            pltpu.VMEM((_LANES,), jnp.float32),
        ],
    )
    def edge_loss(co_hbm, cg_hbm, face_hbm, out_hbm, face_v, bufs, sems,
                  accv):
        wid = lax.axis_index("s") * NC + lax.axis_index("c")
        pltpu.sync_copy(face_hbm, face_v)
        iota = lax.iota(jnp.int32, _LANES)

        vids = []
        for g in range(GROUPS):
            rows = g * _LANES + iota
            vids.append([plsc.load_gather(face_v, [rows * 3 + seg])
                         for seg in range(3)])

        def issue(ci, slot):
            r0 = wid * rows_w + ci * C
            for a, hbm in enumerate((co_hbm, cg_hbm)):
                for p in range(3):
                    pltpu.async_copy(hbm.at[p, pl.ds(r0, C), :],
                                     bufs[slot][a][p], sems[slot])

        def drain(slot):
            for a in range(2):
                for p in range(3):
                    pltpu.make_async_copy(co_hbm.at[0, pl.ds(0, C), :],
                                          bufs[slot][a][p],
                                          sems[slot]).wait()

        def compute(slot, acc):
            def row_body(r, acc):
                rsp = jnp.full((_LANES,), r, jnp.int32)
                for g in range(GROUPS):
                    pts = [[[plsc.load_gather(bufs[slot][a][p],
                                              [rsp, vids[g][s]])
                             for p in range(3)]
                            for s in range(3)]
                           for a in range(2)]
                    for s0, s1 in ((0, 1), (0, 2), (1, 2)):
                        d_o = _edge(pts[0][s0], pts[0][s1])
                        d_g = _edge(pts[1][s0], pts[1][s1])
                        acc = acc + jnp.abs(d_o - d_g)
                return acc

            return lax.fori_loop(0, C, row_body, acc)

        issue(0, 0)

        def pair_body(i2, acc):
            c0 = 2 * i2
            issue(c0 + 1, 1)
            drain(0)
            acc = compute(0, acc)

            @pl.when(c0 + 2 < n_chunks)
            def _():
                issue(c0 + 2, 0)

            drain(1)
            return compute(1, acc)

        acc = lax.fori_loop(0, n_chunks // 2, pair_body,
                            jnp.zeros((_LANES,), jnp.float32))
        accv[...] = acc
        pltpu.sync_copy(accv, out_hbm.at[wid])

    partial = edge_loss(coord_out.transpose(2, 0, 1),
                        coord_gt.transpose(2, 0, 1),
                        face.reshape(-1))
    return jnp.sum(partial) / (B * 3 * Fn)


# speedup vs baseline: 460.9972x; 1.5478x over previous
import functools

import jax
import jax.numpy as jnp
from jax import lax
from jax.experimental import pallas as pl
from jax.experimental.pallas import tpu as pltpu
from jax.experimental.pallas import tpu_sc as plsc

_LANES = 16
_B_SC = 8192
_BS_TC = 512


def _sqrt16(x):
    i = lax.bitcast_convert_type(x, jnp.int32)
    i = 0x5F3759DF - lax.shift_right_logical(i, 1)
    y = lax.bitcast_convert_type(i, jnp.float32)
    y = y * (1.5 - (x * 0.5) * y * y)
    return x * y


def _edge(p, q):
    d0 = p[0] - q[0]
    d1 = p[1] - q[1]
    d2 = p[2] - q[2]
    return _sqrt16(d0 * d0 + d1 * d1 + d2 * d2 + 1e-12)


def _sc_kernel(B_sc, V, Fn):
    RW = 3 * V

    info = plsc.get_sparse_core_info()
    NW = info.num_cores * info.num_subcores
    NC = info.num_cores
    rows_w = B_sc // NW
    C = 16
    n_chunks = rows_w // C
    GROUPS = Fn // _LANES

    mesh = plsc.VectorSubcoreMesh(core_axis_name="c", subcore_axis_name="s")

    @functools.partial(
        pl.kernel,
        mesh=mesh,
        out_type=jax.ShapeDtypeStruct((NW, _LANES), jnp.float32),
        compiler_params=pltpu.CompilerParams(needs_layout_passes=False),
        scratch_types=[
            pltpu.VMEM((Fn * 3,), jnp.int32),
            [[[pltpu.VMEM((C, V), jnp.float32) for _ in range(3)]
              for _ in range(2)]
             for _ in range(2)],
            [pltpu.SemaphoreType.DMA for _ in range(2)],
            pltpu.VMEM((_LANES,), jnp.float32),
        ],
    )
    def edge_loss(co_hbm, cg_hbm, face_hbm, out_hbm, face_v, bufs, sems,
                  accv):
        wid = lax.axis_index("s") * NC + lax.axis_index("c")
        pltpu.sync_copy(face_hbm, face_v)
        iota = lax.iota(jnp.int32, _LANES)

        vids = []
        for g in range(GROUPS):
            rows = g * _LANES + iota
            vids.append([plsc.load_gather(face_v, [rows * 3 + seg])
                         for seg in range(3)])

        def issue(ci, slot):
            r0 = wid * rows_w + ci * C
            for a, hbm in enumerate((co_hbm, cg_hbm)):
                for p in range(3):
                    pltpu.async_copy(hbm.at[p, pl.ds(r0, C), :],
                                     bufs[slot][a][p], sems[slot])

        def drain(slot):
            for a in range(2):
                for p in range(3):
                    pltpu.make_async_copy(co_hbm.at[0, pl.ds(0, C), :],
                                          bufs[slot][a][p],
                                          sems[slot]).wait()

        def compute(slot, acc):
            def row_body(r, acc):
                rsp = jnp.full((_LANES,), r, jnp.int32)
                for g in range(GROUPS):
                    pts = [[[plsc.load_gather(bufs[slot][a][p],
                                              [rsp, vids[g][s]])
                             for p in range(3)]
                            for s in range(3)]
                           for a in range(2)]
                    for s0, s1 in ((0, 1), (0, 2), (1, 2)):
                        d_o = _edge(pts[0][s0], pts[0][s1])
                        d_g = _edge(pts[1][s0], pts[1][s1])
                        acc = acc + jnp.abs(d_o - d_g)
                return acc

            return lax.fori_loop(0, C, row_body, acc)

        issue(0, 0)

        def pair_body(i2, acc):
            c0 = 2 * i2
            issue(c0 + 1, 1)
            drain(0)
            acc = compute(0, acc)

            @pl.when(c0 + 2 < n_chunks)
            def _():
                issue(c0 + 2, 0)

            drain(1)
            return compute(1, acc)

        acc = lax.fori_loop(0, n_chunks // 2, pair_body,
                            jnp.zeros((_LANES,), jnp.float32))
        accv[...] = acc
        pltpu.sync_copy(accv, out_hbm.at[wid])

    return edge_loss


def _tc_kernel(B, B_sc, V, Fn, bs):
    ntc = (B - B_sc) // bs
    E = 3 * Fn

    def body(face_ref, co_ref, cg_ref, out_ref, m_ref):
        i = pl.program_id(0)

        @pl.when(i == 0)
        def _():
            row = lax.broadcasted_iota(jnp.int32, (V, Fn), 0)
            cols = []
            for a, b in ((0, 1), (0, 2), (1, 2)):
                fa = jnp.broadcast_to(face_ref[a:a + 1, :], (V, Fn))
                fb = jnp.broadcast_to(face_ref[b:b + 1, :], (V, Fn))
                cols.append((row == fa).astype(jnp.bfloat16)
                            - (row == fb).astype(jnp.bfloat16))
            m_ref[...] = jnp.concatenate(cols, axis=1)
            out_ref[...] = jnp.zeros_like(out_ref)

        def dists(ref):
            s = None
            for p in range(3):
                xb = ref[p].astype(jnp.bfloat16)
                e = lax.dot_general(xb, m_ref[...], (((1,), (0,)), ((), ())),
                                    preferred_element_type=jnp.float32)
                s = e * e if s is None else s + e * e
            return jnp.sqrt(s + 1e-12)

        diff = jnp.abs(dists(co_ref) - dists(cg_ref))
        out_ref[...] += diff.reshape(bs // 8, 8, E).sum(axis=0)

    return pl.pallas_call(
        body,
        grid=(ntc,),
        in_specs=[
            pl.BlockSpec((3, Fn), lambda i: (0, 0)),
            pl.BlockSpec((3, bs, V), lambda i: (0, B_sc // bs + i, 0)),
            pl.BlockSpec((3, bs, V), lambda i: (0, B_sc // bs + i, 0)),
        ],
        out_specs=pl.BlockSpec((8, E), lambda i: (0, 0)),
        out_shape=jax.ShapeDtypeStruct((8, E), jnp.float32),
        scratch_shapes=[pltpu.VMEM((V, E), jnp.bfloat16)],
    )


def kernel(coord_out, coord_gt, face):
    B, V, _ = coord_out.shape
    Fn = face.shape[0]

    xo = coord_out.transpose(2, 0, 1)
    xg = coord_gt.transpose(2, 0, 1)

    sc_part = _sc_kernel(_B_SC, V, Fn)(xo, xg, face.reshape(-1))
    tc_part = _tc_kernel(B, _B_SC, V, Fn, _BS_TC)(face.T, xo, xg)
    return (jnp.sum(sc_part) + jnp.sum(tc_part)) / (B * 3 * Fn)


# speedup vs baseline: 489.0479x; 1.0608x over previous
import functools

import jax
import jax.numpy as jnp
from jax import lax
from jax.experimental import pallas as pl
from jax.experimental.pallas import tpu as pltpu
from jax.experimental.pallas import tpu_sc as plsc

_LANES = 16
_B_SC = 5120
_BS_TC = 512


def _sqrt16(x):
    i = lax.bitcast_convert_type(x, jnp.int32)
    i = 0x5F3759DF - lax.shift_right_logical(i, 1)
    y = lax.bitcast_convert_type(i, jnp.float32)
    y = y * (1.5 - (x * 0.5) * y * y)
    return x * y


def _edge(p, q):
    d0 = p[0] - q[0]
    d1 = p[1] - q[1]
    d2 = p[2] - q[2]
    return _sqrt16(d0 * d0 + d1 * d1 + d2 * d2 + 1e-12)


def _sc_kernel(B_sc, V, Fn):
    RW = 3 * V

    info = plsc.get_sparse_core_info()
    NW = info.num_cores * info.num_subcores
    NC = info.num_cores
    rows_w = B_sc // NW
    C = 16
    n_chunks = rows_w // C
    GROUPS = Fn // _LANES

    mesh = plsc.VectorSubcoreMesh(core_axis_name="c", subcore_axis_name="s")

    @functools.partial(
        pl.kernel,
        mesh=mesh,
        out_type=jax.ShapeDtypeStruct((NW, _LANES), jnp.float32),
        compiler_params=pltpu.CompilerParams(needs_layout_passes=False),
        scratch_types=[
            pltpu.VMEM((Fn * 3,), jnp.int32),
            [[[pltpu.VMEM((C, V), jnp.float32) for _ in range(3)]
              for _ in range(2)]
             for _ in range(2)],
            [pltpu.SemaphoreType.DMA for _ in range(2)],
            pltpu.VMEM((_LANES,), jnp.float32),
        ],
    )
    def edge_loss(co_hbm, cg_hbm, face_hbm, out_hbm, face_v, bufs, sems,
                  accv):
        wid = lax.axis_index("s") * NC + lax.axis_index("c")
        pltpu.sync_copy(face_hbm, face_v)
        iota = lax.iota(jnp.int32, _LANES)

        vids = []
        for g in range(GROUPS):
            rows = g * _LANES + iota
            vids.append([plsc.load_gather(face_v, [rows * 3 + seg])
                         for seg in range(3)])

        def issue(ci, slot):
            r0 = wid * rows_w + ci * C
            for a, hbm in enumerate((co_hbm, cg_hbm)):
                for p in range(3):
                    pltpu.async_copy(hbm.at[p, pl.ds(r0, C), :],
                                     bufs[slot][a][p], sems[slot])

        def drain(slot):
            for a in range(2):
                for p in range(3):
                    pltpu.make_async_copy(co_hbm.at[0, pl.ds(0, C), :],
                                          bufs[slot][a][p],
                                          sems[slot]).wait()

        def compute(slot, acc):
            def row_body(r, acc):
                rsp = jnp.full((_LANES,), r, jnp.int32)
                for g in range(GROUPS):
                    pts = [[[plsc.load_gather(bufs[slot][a][p],
                                              [rsp, vids[g][s]])
                             for p in range(3)]
                            for s in range(3)]
                           for a in range(2)]
                    for s0, s1 in ((0, 1), (0, 2), (1, 2)):
                        d_o = _edge(pts[0][s0], pts[0][s1])
                        d_g = _edge(pts[1][s0], pts[1][s1])
                        acc = acc + jnp.abs(d_o - d_g)
                return acc

            return lax.fori_loop(0, C, row_body, acc)

        issue(0, 0)

        def pair_body(i2, acc):
            c0 = 2 * i2
            issue(c0 + 1, 1)
            drain(0)
            acc = compute(0, acc)

            @pl.when(c0 + 2 < n_chunks)
            def _():
                issue(c0 + 2, 0)

            drain(1)
            return compute(1, acc)

        acc = lax.fori_loop(0, n_chunks // 2, pair_body,
                            jnp.zeros((_LANES,), jnp.float32))
        accv[...] = acc
        pltpu.sync_copy(accv, out_hbm.at[wid])

    return edge_loss


def _tc_kernel(B, B_sc, V, Fn, bs):
    ntc = (B - B_sc) // bs
    E = 3 * Fn

    def body(face_ref, co_ref, cg_ref, out_ref, m_ref):
        i = pl.program_id(0)

        @pl.when(i == 0)
        def _():
            row = lax.broadcasted_iota(jnp.int32, (V, Fn), 0)
            cols = []
            for a, b in ((0, 1), (0, 2), (1, 2)):
                fa = jnp.broadcast_to(face_ref[a:a + 1, :], (V, Fn))
                fb = jnp.broadcast_to(face_ref[b:b + 1, :], (V, Fn))
                cols.append((row == fa).astype(jnp.bfloat16)
                            - (row == fb).astype(jnp.bfloat16))
            m_ref[...] = jnp.concatenate(cols, axis=1)
            out_ref[...] = jnp.zeros_like(out_ref)

        def dists(ref):
            s = None
            for p in range(3):
                xb = ref[p].astype(jnp.bfloat16)
                e = lax.dot_general(xb, m_ref[...], (((1,), (0,)), ((), ())),
                                    preferred_element_type=jnp.float32)
                s = e * e if s is None else s + e * e
            return jnp.sqrt(s + 1e-12)

        diff = jnp.abs(dists(co_ref) - dists(cg_ref))
        out_ref[...] += diff.reshape(bs // 8, 8, E).sum(axis=0)

    return pl.pallas_call(
        body,
        grid=(ntc,),
        in_specs=[
            pl.BlockSpec((3, Fn), lambda i: (0, 0)),
            pl.BlockSpec((3, bs, V), lambda i: (0, B_sc // bs + i, 0)),
            pl.BlockSpec((3, bs, V), lambda i: (0, B_sc // bs + i, 0)),
        ],
        out_specs=pl.BlockSpec((8, E), lambda i: (0, 0)),
        out_shape=jax.ShapeDtypeStruct((8, E), jnp.float32),
        scratch_shapes=[pltpu.VMEM((V, E), jnp.bfloat16)],
    )


def kernel(coord_out, coord_gt, face):
    B, V, _ = coord_out.shape
    Fn = face.shape[0]

    xo = coord_out.transpose(2, 0, 1)
    xg = coord_gt.transpose(2, 0, 1)

    sc_part = _sc_kernel(_B_SC, V, Fn)(xo, xg, face.reshape(-1))
    tc_part = _tc_kernel(B, _B_SC, V, Fn, _BS_TC)(face.T, xo, xg)
    return (jnp.sum(sc_part) + jnp.sum(tc_part)) / (B * 3 * Fn)


# speedup vs baseline: 513.4970x; 1.0500x over previous
import functools

import jax
import jax.numpy as jnp
from jax import lax
from jax.experimental import pallas as pl
from jax.experimental.pallas import tpu as pltpu
from jax.experimental.pallas import tpu_sc as plsc

_LANES = 16
_B_SC = 6144
_BS_TC = 512


def _sqrt16(x):
    i = lax.bitcast_convert_type(x, jnp.int32)
    i = 0x5F3759DF - lax.shift_right_logical(i, 1)
    y = lax.bitcast_convert_type(i, jnp.float32)
    y = y * (1.5 - (x * 0.5) * y * y)
    return x * y


def _edge(p, q):
    d0 = p[0] - q[0]
    d1 = p[1] - q[1]
    d2 = p[2] - q[2]
    return _sqrt16(d0 * d0 + d1 * d1 + d2 * d2 + 1e-12)


def _sc_kernel(B_sc, V, Fn):
    RW = 3 * V

    info = plsc.get_sparse_core_info()
    NW = info.num_cores * info.num_subcores
    NC = info.num_cores
    rows_w = B_sc // NW
    C = 16
    n_chunks = rows_w // C
    GROUPS = Fn // _LANES

    mesh = plsc.VectorSubcoreMesh(core_axis_name="c", subcore_axis_name="s")

    @functools.partial(
        pl.kernel,
        mesh=mesh,
        out_type=jax.ShapeDtypeStruct((NW, _LANES), jnp.float32),
        compiler_params=pltpu.CompilerParams(needs_layout_passes=False),
        scratch_types=[
            pltpu.VMEM((Fn * 3,), jnp.int32),
            [[[pltpu.VMEM((C, V), jnp.float32) for _ in range(3)]
              for _ in range(2)]
             for _ in range(2)],
            [pltpu.SemaphoreType.DMA for _ in range(2)],
            pltpu.VMEM((_LANES,), jnp.float32),
        ],
    )
    def edge_loss(co_hbm, cg_hbm, face_hbm, out_hbm, face_v, bufs, sems,
                  accv):
        wid = lax.axis_index("s") * NC + lax.axis_index("c")
        pltpu.sync_copy(face_hbm, face_v)
        iota = lax.iota(jnp.int32, _LANES)

        vids = []
        for g in range(GROUPS):
            rows = g * _LANES + iota
            vids.append([plsc.load_gather(face_v, [rows * 3 + seg])
                         for seg in range(3)])

        def issue(ci, slot):
            r0 = wid * rows_w + ci * C
            for a, hbm in enumerate((co_hbm, cg_hbm)):
                for p in range(3):
                    pltpu.async_copy(hbm.at[p, pl.ds(r0, C), :],
                                     bufs[slot][a][p], sems[slot])

        def drain(slot):
            for a in range(2):
                for p in range(3):
                    pltpu.make_async_copy(co_hbm.at[0, pl.ds(0, C), :],
                                          bufs[slot][a][p],
                                          sems[slot]).wait()

        def compute(slot, acc):
            def row_body(r, acc):
                rsp = jnp.full((_LANES,), r, jnp.int32)
                for g in range(GROUPS):
                    pts = [[[plsc.load_gather(bufs[slot][a][p],
                                              [rsp, vids[g][s]])
                             for p in range(3)]
                            for s in range(3)]
                           for a in range(2)]
                    for s0, s1 in ((0, 1), (0, 2), (1, 2)):
                        d_o = _edge(pts[0][s0], pts[0][s1])
                        d_g = _edge(pts[1][s0], pts[1][s1])
                        acc = acc + jnp.abs(d_o - d_g)
                return acc

            return lax.fori_loop(0, C, row_body, acc)

        issue(0, 0)

        def pair_body(i2, acc):
            c0 = 2 * i2
            issue(c0 + 1, 1)
            drain(0)
            acc = compute(0, acc)

            @pl.when(c0 + 2 < n_chunks)
            def _():
                issue(c0 + 2, 0)

            drain(1)
            return compute(1, acc)

        acc = lax.fori_loop(0, n_chunks // 2, pair_body,
                            jnp.zeros((_LANES,), jnp.float32))
        accv[...] = acc
        pltpu.sync_copy(accv, out_hbm.at[wid])

    return edge_loss


def _tc_kernel(B, B_sc, V, Fn, bs):
    ntc = (B - B_sc) // bs
    E = 3 * Fn

    def body(face_ref, co_ref, cg_ref, out_ref, m_ref):
        i = pl.program_id(0)

        @pl.when(i == 0)
        def _():
            row = lax.broadcasted_iota(jnp.int32, (V, Fn), 0)
            cols = []
            for a, b in ((0, 1), (0, 2), (1, 2)):
                fa = jnp.broadcast_to(face_ref[a:a + 1, :], (V, Fn))
                fb = jnp.broadcast_to(face_ref[b:b + 1, :], (V, Fn))
                cols.append((row == fa).astype(jnp.bfloat16)
                            - (row == fb).astype(jnp.bfloat16))
            m_ref[...] = jnp.concatenate(cols, axis=1)
            out_ref[...] = jnp.zeros_like(out_ref)

        def dists(ref):
            s = None
            for p in range(3):
                xb = ref[p].astype(jnp.bfloat16)
                e = lax.dot_general(xb, m_ref[...], (((1,), (0,)), ((), ())),
                                    preferred_element_type=jnp.float32)
                s = e * e if s is None else s + e * e
            return jnp.sqrt(s + 1e-12)

        diff = jnp.abs(dists(co_ref) - dists(cg_ref))
        out_ref[...] += diff.reshape(bs // 8, 8, E).sum(axis=0)

    return pl.pallas_call(
        body,
        grid=(ntc,),
        in_specs=[
            pl.BlockSpec((3, Fn), lambda i: (0, 0)),
            pl.BlockSpec((3, bs, V), lambda i: (0, B_sc // bs + i, 0)),
            pl.BlockSpec((3, bs, V), lambda i: (0, B_sc // bs + i, 0)),
        ],
        out_specs=pl.BlockSpec((8, E), lambda i: (0, 0)),
        out_shape=jax.ShapeDtypeStruct((8, E), jnp.float32),
        scratch_shapes=[pltpu.VMEM((V, E), jnp.bfloat16)],
    )


def kernel(coord_out, coord_gt, face):
    B, V, _ = coord_out.shape
    Fn = face.shape[0]

    xo = coord_out.transpose(2, 0, 1)
    xg = coord_gt.transpose(2, 0, 1)

    sc_part = _sc_kernel(_B_SC, V, Fn)(xo, xg, face.reshape(-1))
    tc_part = _tc_kernel(B, _B_SC, V, Fn, _BS_TC)(face.T, xo, xg)
    return (jnp.sum(sc_part) + jnp.sum(tc_part)) / (B * 3 * Fn)
